# Initial kernel scaffold; baseline (speedup 1.0000x reference)
import jax
import jax.numpy as jnp
from jax.experimental import pallas as pl


def kernel(x, edge_index, batch, mlp1_w1, mlp1_b1, mlp1_w2, mlp1_b2, gbn1_g, gbn1_b, mlp2_w1, mlp2_b1, mlp2_w2, mlp2_b2, gbn2_g, gbn2_b, mlp3_w, mlp3_b, gbn3_g, gbn3_b, lin1_w, lin1_b, fbn1_g, fbn1_b, lin2_w, lin2_b, fbn2_g, fbn2_b, lin3_w, lin3_b):
    raise NotImplementedError("write your pallas kernel here")


# speedup vs baseline: 4.4396x; 4.4396x over previous
import functools

import jax
import jax.numpy as jnp
from jax import lax
from jax.experimental import pallas as pl
from jax.experimental.pallas import tpu as pltpu

N = 4288
B = 16
ROI = 268
C = 32
K = 32
E = 68608
SLOPE = 0.33
EPS = 1e-5

RB = 536
NRB = N // RB
EB = 4288
NEB = E // EB
UNROLL = 8
KNN_NODES = 64
KNB = N // KNN_NODES

MASK_INF = 1e30
PICK_INF = 2e30


def _lrelu(x):
    return jnp.where(x >= 0, x, SLOPE * x)


def _bn(x, g, b):
    mu = jnp.mean(x, axis=0, keepdims=True)
    var = jnp.mean((x - mu) ** 2, axis=0, keepdims=True)
    return (x - mu) * jax.lax.rsqrt(var + EPS) * g + b


def _proj_body(x_ref, w_ref, b_ref, p_ref, sq_ref):
    x = x_ref[...]
    p_ref[...] = jnp.dot(x, w_ref[...], preferred_element_type=jnp.float32) + b_ref[...]
    sq_ref[...] = jnp.sum(x * x, axis=1, keepdims=True)


def _proj(x, wfull, bfull):
    return pl.pallas_call(
        _proj_body,
        out_shape=(jax.ShapeDtypeStruct((N, 160), jnp.float32),
                   jax.ShapeDtypeStruct((N, 1), jnp.float32)),
    )(x, wfull, bfull)


def _dist_body(x_ref, xt_ref, sqr_ref, sqc_ref, br_ref, bc_ref, idx_ref, d_ref):
    xb = x_ref[...]
    d = sqr_ref[...] + sqc_ref[...] - 2.0 * jnp.dot(
        xb, xt_ref[...], preferred_element_type=jnp.float32)
    d = jnp.where(br_ref[...] == bc_ref[...], d, MASK_INF)
    d_ref[...] = d
    col = lax.broadcasted_iota(jnp.int32, (RB, N), 1)

    def body(k, acc):
        dcur = d_ref[...]
        m = jnp.min(dcur, axis=1, keepdims=True)
        cand = jnp.where(dcur == m, col, N)
        sel = jnp.min(cand, axis=1, keepdims=True)
        acc = jnp.where(lax.broadcasted_iota(jnp.int32, (RB, K), 1) == k, sel, acc)
        d_ref[...] = jnp.where(col == sel, PICK_INF, dcur)
        return acc

    idx_ref[...] = lax.fori_loop(0, K, body, jnp.zeros((RB, K), jnp.int32))


def _dist_topk(x, xt, sq, sqt, batch_col, batch_row):
    return pl.pallas_call(
        _dist_body,
        grid=(NRB,),
        in_specs=[
            pl.BlockSpec((RB, ROI), lambda i: (i, 0)),
            pl.BlockSpec((ROI, N), lambda i: (0, 0)),
            pl.BlockSpec((RB, 1), lambda i: (i, 0)),
            pl.BlockSpec((1, N), lambda i: (0, 0)),
            pl.BlockSpec((RB, 1), lambda i: (i, 0)),
            pl.BlockSpec((1, N), lambda i: (0, 0)),
        ],
        out_specs=pl.BlockSpec((RB, K), lambda i: (i, 0)),
        out_shape=jax.ShapeDtypeStruct((N, K), jnp.int32),
        scratch_shapes=[pltpu.VMEM((RB, N), jnp.float32)],
    )(x, xt, sq, sqt, batch_col, batch_row)


def _scatter_body(dst_sm, pa_ref, pb_ref, w_ref, b_ref, out_ref, s_ref, *accs):
    i = pl.program_id(0)

    @pl.when(i == 0)
    def _():
        for a in accs:
            a[...] = jnp.full((N, 48), -jnp.inf, jnp.float32)

    pre = pa_ref[...] + pb_ref[...]
    lane = lax.broadcasted_iota(jnp.int32, (EB, 48), 1)
    t = jnp.where(lane < 32, _lrelu(pre), pre)
    s_ref[...] = jnp.dot(t, w_ref[...], preferred_element_type=jnp.float32) + b_ref[...]
    base = i * EB

    def body(r, carry):
        r8 = r * UNROLL
        for u in range(UNROLL):
            d = dst_sm[base + r8 + u]
            row = s_ref[pl.ds(r8 + u, 1), :]
            accs[u][pl.ds(d, 1), :] = jnp.maximum(accs[u][pl.ds(d, 1), :], row)
        return carry

    lax.fori_loop(0, EB // UNROLL, body, 0)

    @pl.when(i == NEB - 1)
    def _():
        m = accs[0][...]
        for a in accs[1:]:
            m = jnp.maximum(m, a[...])
        out_ref[...] = jnp.where(jnp.isfinite(m), m, 0.0)


def _edge_scatter(dst, pa, pb, w2ext, b2ext):
    grid_spec = pltpu.PrefetchScalarGridSpec(
        num_scalar_prefetch=1,
        grid=(NEB,),
        in_specs=[
            pl.BlockSpec((EB, 48), lambda i, *_: (i, 0)),
            pl.BlockSpec((EB, 48), lambda i, *_: (i, 0)),
            pl.BlockSpec((48, 48), lambda i, *_: (0, 0)),
            pl.BlockSpec((1, 48), lambda i, *_: (0, 0)),
        ],
        out_specs=pl.BlockSpec((N, 48), lambda i, *_: (0, 0)),
        scratch_shapes=[pltpu.VMEM((EB, 48), jnp.float32)]
        + [pltpu.VMEM((N, 48), jnp.float32) for _ in range(UNROLL)],
    )
    return pl.pallas_call(
        _scatter_body,
        grid_spec=grid_spec,
        out_shape=jax.ShapeDtypeStruct((N, 48), jnp.float32),
        compiler_params=pltpu.CompilerParams(
            dimension_semantics=("arbitrary",)),
    )(dst, pa, pb, w2ext, b2ext)


def _knn_body(b2g_ref, a2_ref, w_ref, b_ref, out_ref):
    b2 = b2g_ref[...]
    a2 = a2_ref[...]
    a2r = jnp.broadcast_to(a2[:, None, :], (KNN_NODES, K, C)).reshape(KNN_NODES * K, C)
    m = jnp.dot(_lrelu(b2 + a2r), w_ref[...],
                preferred_element_type=jnp.float32) + b_ref[...]
    out_ref[...] = jnp.max(m.reshape(KNN_NODES, K, C), axis=1)


def _knn_max(b2g, a2, w, b):
    return pl.pallas_call(
        _knn_body,
        grid=(KNB,),
        in_specs=[
            pl.BlockSpec((KNN_NODES * K, C), lambda i: (i, 0)),
            pl.BlockSpec((KNN_NODES, C), lambda i: (i, 0)),
            pl.BlockSpec((C, C), lambda i: (0, 0)),
            pl.BlockSpec((1, C), lambda i: (0, 0)),
        ],
        out_specs=pl.BlockSpec((KNN_NODES, C), lambda i: (i, 0)),
        out_shape=jax.ShapeDtypeStruct((N, C), jnp.float32),
    )(b2g, a2, w, b)


def _final1_body(acc_ref, x2_ref, bc_ref,
                 g1_ref, b1_ref, g2_ref, b2_ref, g3_ref, b3_ref,
                 p1_ref, p2_ref, x3_ref):
    accm = acc_ref[...]
    x1 = _lrelu(_bn(accm[:, :32], g1_ref[...], b1_ref[...]))
    x2 = _lrelu(_bn(x2_ref[...], g2_ref[...], b2_ref[...]))
    x3 = _lrelu(_bn(accm[:, 32:33], g3_ref[...], b3_ref[...]))
    oh = (lax.broadcasted_iota(jnp.int32, (B, N), 0) == bc_ref[...]).astype(jnp.float32)
    counts = jnp.maximum(jnp.sum(oh, axis=1, keepdims=True), 1.0)
    p1_ref[...] = jnp.dot(oh, x1, preferred_element_type=jnp.float32) / counts
    p2_ref[...] = jnp.dot(oh, x2, preferred_element_type=jnp.float32) / counts
    x3_ref[...] = x3


def _final1(acc, x2raw, batch_row, g1, b1, g2, b2, g3, b3):
    return pl.pallas_call(
        _final1_body,
        out_shape=(jax.ShapeDtypeStruct((B, C), jnp.float32),
                   jax.ShapeDtypeStruct((B, C), jnp.float32),
                   jax.ShapeDtypeStruct((N, 1), jnp.float32)),
    )(acc, x2raw, batch_row, g1, b1, g2, b2, g3, b3)


def _final2_body(h_ref, w1_ref, b1_ref, g1_ref, bb1_ref,
                 w2_ref, b2_ref, g2_ref, bb2_ref, w3_ref, b3_ref, out_ref):
    h = h_ref[...]
    h = _lrelu(_bn(jnp.dot(h, w1_ref[...], preferred_element_type=jnp.float32)
                   + b1_ref[...], g1_ref[...], bb1_ref[...]))
    h = _lrelu(_bn(jnp.dot(h, w2_ref[...], preferred_element_type=jnp.float32)
                   + b2_ref[...], g2_ref[...], bb2_ref[...]))
    out_ref[...] = jnp.dot(h, w3_ref[...], preferred_element_type=jnp.float32) + b3_ref[...]


def _final2(h, w1, b1, g1, bb1, w2, b2, g2, bb2, w3, b3):
    return pl.pallas_call(
        _final2_body,
        out_shape=jax.ShapeDtypeStruct((B, 1), jnp.float32),
    )(h, w1, b1, g1, bb1, w2, b2, g2, bb2, w3, b3)


def kernel(x, edge_index, batch, mlp1_w1, mlp1_b1, mlp1_w2, mlp1_b2, gbn1_g, gbn1_b,
           mlp2_w1, mlp2_b1, mlp2_w2, mlp2_b2, gbn2_g, gbn2_b,
           mlp3_w, mlp3_b, gbn3_g, gbn3_b,
           lin1_w, lin1_b, fbn1_g, fbn1_b, lin2_w, lin2_b, fbn2_g, fbn2_b,
           lin3_w, lin3_b):
    f32 = jnp.float32
    src = edge_index[0]
    dst = edge_index[1]

    z15 = jnp.zeros((ROI, 15), f32)
    wfull = jnp.concatenate([
        mlp1_w1[:ROI] - mlp1_w1[ROI:],
        mlp3_w[:ROI] - mlp3_w[ROI:],
        z15,
        mlp1_w1[ROI:],
        mlp3_w[ROI:],
        z15,
        mlp2_w1[:ROI] - mlp2_w1[ROI:],
        mlp2_w1[ROI:],
    ], axis=1)
    bfull = jnp.concatenate([
        mlp1_b1, mlp3_b, jnp.zeros((15,), f32),
        jnp.zeros((48,), f32),
        mlp2_b1, jnp.zeros((32,), f32),
    ])[None, :]
    w2ext = jnp.zeros((48, 48), f32)
    w2ext = w2ext.at[:32, :32].set(mlp1_w2)
    w2ext = w2ext.at[32:, 32:].set(jnp.eye(16, dtype=f32))
    b2ext = jnp.concatenate([mlp1_b2, jnp.zeros((16,), f32)])[None, :]

    p, sq = _proj(x, wfull, bfull)
    ta = p[:, :48]
    tb = p[:, 48:96]
    a2 = p[:, 96:128]
    b2t = p[:, 128:160]

    batch_row = batch[None, :].astype(jnp.int32)
    idx = _dist_topk(x, x.T, sq, sq.T, batch[:, None].astype(jnp.int32), batch_row)

    pa = ta[dst]
    pb = tb[src]
    b2g = b2t[idx.reshape(-1)]

    acc = _edge_scatter(dst, pa, pb, w2ext, b2ext)

    x2raw = _knn_max(b2g, a2, mlp2_w2, mlp2_b2[None, :])

    p1, p2, x3 = _final1(acc, x2raw, batch_row,
                         gbn1_g[None, :], gbn1_b[None, :],
                         gbn2_g[None, :], gbn2_b[None, :],
                         gbn3_g[None, :], gbn3_b[None, :])

    h0 = jnp.concatenate([p1, p2, x3.reshape(B, ROI)], axis=1)

    return _final2(h0, lin1_w, lin1_b[None, :], fbn1_g[None, :], fbn1_b[None, :],
                   lin2_w, lin2_b[None, :], fbn2_g[None, :], fbn2_b[None, :],
                   lin3_w, lin3_b[None, :])


# speedup vs baseline: 7.5262x; 1.6952x over previous
import functools

import jax
import jax.numpy as jnp
from jax import lax
from jax.experimental import pallas as pl
from jax.experimental.pallas import tpu as pltpu
from jax.experimental.pallas import tpu_sc as plsc

N = 4288
B = 16
ROI = 268
C = 32
K = 32
E = 68608
SLOPE = 0.33
EPS = 1e-5

RB = 536
NRB = N // RB
EB = 4288
NEB = E // EB
UNROLL = 8
KNN_NODES = 64
KNB = N // KNN_NODES

MASK_INF = 1e30
PICK_INF = 2e30


def _lrelu(x):
    return jnp.where(x >= 0, x, SLOPE * x)


def _bn(x, g, b):
    mu = jnp.mean(x, axis=0, keepdims=True)
    var = jnp.mean((x - mu) ** 2, axis=0, keepdims=True)
    return (x - mu) * jax.lax.rsqrt(var + EPS) * g + b


def _proj_body(x_ref, w_ref, b_ref, w2_ref, b2_ref, p_ref, a2_ref, sq_ref):
    x = x_ref[...]
    p_ref[...] = jnp.dot(x, w_ref[...], preferred_element_type=jnp.float32) + b_ref[...]
    a2_ref[...] = jnp.dot(x, w2_ref[...], preferred_element_type=jnp.float32) + b2_ref[...]
    sq_ref[...] = jnp.sum(x * x, axis=1, keepdims=True)


def _proj(x, w128, b128, wa2, ba2):
    return pl.pallas_call(
        _proj_body,
        out_shape=(jax.ShapeDtypeStruct((N, 128), jnp.float32),
                   jax.ShapeDtypeStruct((N, C), jnp.float32),
                   jax.ShapeDtypeStruct((N, 1), jnp.float32)),
    )(x, w128, b128, wa2, ba2)


def _dist_body(x_ref, xt_ref, sqr_ref, sqc_ref, br_ref, bc_ref, idx_ref, d_ref):
    xb = x_ref[...]
    d = sqr_ref[...] + sqc_ref[...] - 2.0 * jnp.dot(
        xb, xt_ref[...], preferred_element_type=jnp.float32)
    d = jnp.where(br_ref[...] == bc_ref[...], d, MASK_INF)
    d_ref[...] = d
    col = lax.broadcasted_iota(jnp.int32, (RB, N), 1)

    def body(k, acc):
        dcur = d_ref[...]
        m = jnp.min(dcur, axis=1, keepdims=True)
        cand = jnp.where(dcur == m, col, N)
        sel = jnp.min(cand, axis=1, keepdims=True)
        acc = jnp.where(lax.broadcasted_iota(jnp.int32, (RB, K), 1) == k, sel, acc)
        d_ref[...] = jnp.where(col == sel, PICK_INF, dcur)
        return acc

    idx_ref[...] = lax.fori_loop(0, K, body, jnp.zeros((RB, K), jnp.int32))


def _dist_topk(x, xt, sq, sqt, batch_col, batch_row):
    return pl.pallas_call(
        _dist_body,
        grid=(NRB,),
        in_specs=[
            pl.BlockSpec((RB, ROI), lambda i: (i, 0)),
            pl.BlockSpec((ROI, N), lambda i: (0, 0)),
            pl.BlockSpec((RB, 1), lambda i: (i, 0)),
            pl.BlockSpec((1, N), lambda i: (0, 0)),
            pl.BlockSpec((RB, 1), lambda i: (i, 0)),
            pl.BlockSpec((1, N), lambda i: (0, 0)),
        ],
        out_specs=pl.BlockSpec((RB, K), lambda i: (i, 0)),
        out_shape=jax.ShapeDtypeStruct((N, K), jnp.int32),
        scratch_shapes=[pltpu.VMEM((RB, N), jnp.float32)],
    )(x, xt, sq, sqt, batch_col, batch_row)


def _scatter_body(dst_sm, pa_ref, pb_ref, w_ref, b_ref, out_ref, s_ref, *accs):
    i = pl.program_id(0)

    @pl.when(i == 0)
    def _():
        for a in accs:
            a[...] = jnp.full((N, 48), -jnp.inf, jnp.float32)

    pre = pa_ref[:, 0:48] + pb_ref[:, 48:96]
    lane = lax.broadcasted_iota(jnp.int32, (EB, 48), 1)
    t = jnp.where(lane < 32, _lrelu(pre), pre)
    s_ref[...] = jnp.dot(t, w_ref[...], preferred_element_type=jnp.float32) + b_ref[...]
    base = i * EB

    def body(r, carry):
        r8 = r * UNROLL
        for u in range(UNROLL):
            d = dst_sm[base + r8 + u]
            row = s_ref[pl.ds(r8 + u, 1), :]
            accs[u][pl.ds(d, 1), :] = jnp.maximum(accs[u][pl.ds(d, 1), :], row)
        return carry

    lax.fori_loop(0, EB // UNROLL, body, 0)

    @pl.when(i == NEB - 1)
    def _():
        m = accs[0][...]
        for a in accs[1:]:
            m = jnp.maximum(m, a[...])
        out_ref[...] = jnp.where(jnp.isfinite(m), m, 0.0)


def _edge_scatter(dst, pa, pb, w2ext, b2ext):
    grid_spec = pltpu.PrefetchScalarGridSpec(
        num_scalar_prefetch=1,
        grid=(NEB,),
        in_specs=[
            pl.BlockSpec((EB, 128), lambda i, *_: (i, 0)),
            pl.BlockSpec((EB, 128), lambda i, *_: (i, 0)),
            pl.BlockSpec((48, 48), lambda i, *_: (0, 0)),
            pl.BlockSpec((1, 48), lambda i, *_: (0, 0)),
        ],
        out_specs=pl.BlockSpec((N, 48), lambda i, *_: (0, 0)),
        scratch_shapes=[pltpu.VMEM((EB, 48), jnp.float32)]
        + [pltpu.VMEM((N, 48), jnp.float32) for _ in range(UNROLL)],
    )
    return pl.pallas_call(
        _scatter_body,
        grid_spec=grid_spec,
        out_shape=jax.ShapeDtypeStruct((N, 48), jnp.float32),
        compiler_params=pltpu.CompilerParams(
            dimension_semantics=("arbitrary",)),
    )(dst, pa, pb, w2ext, b2ext)


def _knn_body(b2g_ref, a2_ref, w_ref, b_ref, out_ref):
    b2 = b2g_ref[:, 96:128]
    a2 = a2_ref[...]
    a2r = jnp.broadcast_to(a2[:, None, :], (KNN_NODES, K, C)).reshape(KNN_NODES * K, C)
    m = jnp.dot(_lrelu(b2 + a2r), w_ref[...],
                preferred_element_type=jnp.float32) + b_ref[...]
    out_ref[...] = jnp.max(m.reshape(KNN_NODES, K, C), axis=1)


def _knn_max(b2g, a2, w, b):
    return pl.pallas_call(
        _knn_body,
        grid=(KNB,),
        in_specs=[
            pl.BlockSpec((KNN_NODES * K, 128), lambda i: (i, 0)),
            pl.BlockSpec((KNN_NODES, C), lambda i: (i, 0)),
            pl.BlockSpec((C, C), lambda i: (0, 0)),
            pl.BlockSpec((1, C), lambda i: (0, 0)),
        ],
        out_specs=pl.BlockSpec((KNN_NODES, C), lambda i: (i, 0)),
        out_shape=jax.ShapeDtypeStruct((N, C), jnp.float32),
    )(b2g, a2, w, b)


def _final1_body(acc_ref, x2_ref, bc_ref,
                 g1_ref, b1_ref, g2_ref, b2_ref, g3_ref, b3_ref,
                 p1_ref, p2_ref, x3_ref):
    accm = acc_ref[...]
    x1 = _lrelu(_bn(accm[:, :32], g1_ref[...], b1_ref[...]))
    x2 = _lrelu(_bn(x2_ref[...], g2_ref[...], b2_ref[...]))
    x3 = _lrelu(_bn(accm[:, 32:33], g3_ref[...], b3_ref[...]))
    oh = (lax.broadcasted_iota(jnp.int32, (B, N), 0) == bc_ref[...]).astype(jnp.float32)
    counts = jnp.maximum(jnp.sum(oh, axis=1, keepdims=True), 1.0)
    p1_ref[...] = jnp.dot(oh, x1, preferred_element_type=jnp.float32) / counts
    p2_ref[...] = jnp.dot(oh, x2, preferred_element_type=jnp.float32) / counts
    x3_ref[...] = x3


def _final1(acc, x2raw, batch_row, g1, b1, g2, b2, g3, b3):
    return pl.pallas_call(
        _final1_body,
        out_shape=(jax.ShapeDtypeStruct((B, C), jnp.float32),
                   jax.ShapeDtypeStruct((B, C), jnp.float32),
                   jax.ShapeDtypeStruct((N, 1), jnp.float32)),
    )(acc, x2raw, batch_row, g1, b1, g2, b2, g3, b3)


def _final2_body(h_ref, w1_ref, b1_ref, g1_ref, bb1_ref,
                 w2_ref, b2_ref, g2_ref, bb2_ref, w3_ref, b3_ref, out_ref):
    h = h_ref[...]
    h = _lrelu(_bn(jnp.dot(h, w1_ref[...], preferred_element_type=jnp.float32)
                   + b1_ref[...], g1_ref[...], bb1_ref[...]))
    h = _lrelu(_bn(jnp.dot(h, w2_ref[...], preferred_element_type=jnp.float32)
                   + b2_ref[...], g2_ref[...], bb2_ref[...]))
    out_ref[...] = jnp.dot(h, w3_ref[...], preferred_element_type=jnp.float32) + b3_ref[...]


def _final2(h, w1, b1, g1, bb1, w2, b2, g2, bb2, w3, b3):
    return pl.pallas_call(
        _final2_body,
        out_shape=jax.ShapeDtypeStruct((B, 1), jnp.float32),
    )(h, w1, b1, g1, bb1, w2, b2, g2, bb2, w3, b3)


NW = 32
EPW = E // NW
ECH = EPW // 4
KPW = (N * K) // NW
KCH = KPW // 8


def _sc_gather(t128, dst, src, idxk):
    mesh = plsc.VectorSubcoreMesh(core_axis_name="c", subcore_axis_name="s")

    @functools.partial(
        pl.kernel, mesh=mesh,
        out_type=[jax.ShapeDtypeStruct((E, 128), jnp.float32),
                  jax.ShapeDtypeStruct((E, 128), jnp.float32),
                  jax.ShapeDtypeStruct((N * K, 128), jnp.float32)],
        scratch_types=[
            pltpu.VMEM((ECH,), jnp.int32),
            pltpu.VMEM((ECH, 128), jnp.float32),
            pltpu.SemaphoreType.DMA,
        ],
    )
    def gather_k(t_h, dst_h, src_h, idxk_h,
                 pa_h, pb_h, b2g_h, idx_v, rows_v, sem):
        wid = lax.axis_index("s") * 2 + lax.axis_index("c")
        for ch in range(4):
            base = wid * EPW + ch * ECH
            pltpu.sync_copy(dst_h.at[pl.ds(base, ECH)], idx_v)
            pltpu.async_copy(t_h.at[idx_v], rows_v, sem).wait()
            pltpu.sync_copy(rows_v, pa_h.at[pl.ds(base, ECH)])
            pltpu.sync_copy(src_h.at[pl.ds(base, ECH)], idx_v)
            pltpu.async_copy(t_h.at[idx_v], rows_v, sem).wait()
            pltpu.sync_copy(rows_v, pb_h.at[pl.ds(base, ECH)])
        for ch in range(8):
            base = wid * KPW + ch * KCH
            pltpu.sync_copy(idxk_h.at[pl.ds(base, KCH)], idx_v)
            pltpu.async_copy(t_h.at[idx_v], rows_v, sem).wait()
            pltpu.sync_copy(rows_v, b2g_h.at[pl.ds(base, KCH)])

    return gather_k(t128, dst, src, idxk)


def kernel(x, edge_index, batch, mlp1_w1, mlp1_b1, mlp1_w2, mlp1_b2, gbn1_g, gbn1_b,
           mlp2_w1, mlp2_b1, mlp2_w2, mlp2_b2, gbn2_g, gbn2_b,
           mlp3_w, mlp3_b, gbn3_g, gbn3_b,
           lin1_w, lin1_b, fbn1_g, fbn1_b, lin2_w, lin2_b, fbn2_g, fbn2_b,
           lin3_w, lin3_b):
    f32 = jnp.float32
    src = edge_index[0]
    dst = edge_index[1]

    z15 = jnp.zeros((ROI, 15), f32)
    w128 = jnp.concatenate([
        mlp1_w1[:ROI] - mlp1_w1[ROI:],
        mlp3_w[:ROI] - mlp3_w[ROI:],
        z15,
        mlp1_w1[ROI:],
        mlp3_w[ROI:],
        z15,
        mlp2_w1[ROI:],
    ], axis=1)
    b128 = jnp.concatenate([
        mlp1_b1, mlp3_b, jnp.zeros((15,), f32),
        jnp.zeros((48,), f32),
        jnp.zeros((32,), f32),
    ])[None, :]
    wa2 = mlp2_w1[:ROI] - mlp2_w1[ROI:]
    ba2 = mlp2_b1[None, :]
    w2ext = jnp.zeros((48, 48), f32)
    w2ext = w2ext.at[:32, :32].set(mlp1_w2)
    w2ext = w2ext.at[32:48, 32:].set(jnp.eye(16, dtype=f32))
    b2ext = jnp.concatenate([mlp1_b2, jnp.zeros((16,), f32)])[None, :]

    t128, a2, sq = _proj(x, w128, b128, wa2, ba2)

    batch_row = batch[None, :].astype(jnp.int32)
    idx = _dist_topk(x, x.T, sq, sq.T, batch[:, None].astype(jnp.int32), batch_row)

    pa, pb, b2g = _sc_gather(t128, dst, src, idx.reshape(-1))

    acc = _edge_scatter(dst, pa, pb, w2ext, b2ext)

    x2raw = _knn_max(b2g, a2, mlp2_w2, mlp2_b2[None, :])

    p1, p2, x3 = _final1(acc, x2raw, batch_row,
                         gbn1_g[None, :], gbn1_b[None, :],
                         gbn2_g[None, :], gbn2_b[None, :],
                         gbn3_g[None, :], gbn3_b[None, :])

    h0 = jnp.concatenate([p1, p2, x3.reshape(B, ROI)], axis=1)

    return _final2(h0, lin1_w, lin1_b[None, :], fbn1_g[None, :], fbn1_b[None, :],
                   lin2_w, lin2_b[None, :], fbn2_g[None, :], fbn2_b[None, :],
                   lin3_w, lin3_b[None, :])


# speedup vs baseline: 7.7872x; 1.0347x over previous
import functools

import jax
import jax.numpy as jnp
from jax import lax
from jax.experimental import pallas as pl
from jax.experimental.pallas import tpu as pltpu
from jax.experimental.pallas import tpu_sc as plsc

N = 4288
B = 16
ROI = 268
C = 32
K = 32
E = 68608
SLOPE = 0.33
EPS = 1e-5

RB = 536
NRB = N // RB
EB = 2144
NEB = E // EB
UNROLL = 16
KNN_NODES = 64
KNB = N // KNN_NODES

MASK_INF = 1e30
PICK_INF = 2e30


def _lrelu(x):
    return jnp.where(x >= 0, x, SLOPE * x)


def _bn(x, g, b):
    mu = jnp.mean(x, axis=0, keepdims=True)
    var = jnp.mean((x - mu) ** 2, axis=0, keepdims=True)
    return (x - mu) * jax.lax.rsqrt(var + EPS) * g + b


def _proj_body(x_ref, w_ref, b_ref, w2_ref, b2_ref, p_ref, a2_ref):
    x = x_ref[...]
    p_ref[...] = jnp.dot(x, w_ref[...], precision=lax.Precision.HIGHEST,
                         preferred_element_type=jnp.float32) + b_ref[...]
    a2_ref[...] = jnp.dot(x, w2_ref[...], precision=lax.Precision.HIGHEST,
                          preferred_element_type=jnp.float32) + b2_ref[...]


def _proj(x, w128, b128, wa2, ba2):
    return pl.pallas_call(
        _proj_body,
        out_shape=(jax.ShapeDtypeStruct((N, 128), jnp.float32),
                   jax.ShapeDtypeStruct((N, C), jnp.float32)),
    )(x, w128, b128, wa2, ba2)


def _dist_body(x_ref, xt_ref, sqr_ref, sqc_ref, br_ref, bc_ref, idx_ref, d_ref):
    xb = x_ref[...]
    d = sqr_ref[...] + sqc_ref[...] - 2.0 * jnp.dot(
        xb, xt_ref[...], preferred_element_type=jnp.float32)
    d = jnp.where(br_ref[...] == bc_ref[...], d, MASK_INF)
    d_ref[...] = d
    col = lax.broadcasted_iota(jnp.int32, (RB, N), 1)

    def body(k, acc):
        dcur = d_ref[...]
        m = jnp.min(dcur, axis=1, keepdims=True)
        cand = jnp.where(dcur == m, col, N)
        sel = jnp.min(cand, axis=1, keepdims=True)
        acc = jnp.where(lax.broadcasted_iota(jnp.int32, (RB, K), 1) == k, sel, acc)
        d_ref[...] = jnp.where(col == sel, PICK_INF, dcur)
        return acc

    idx_ref[...] = lax.fori_loop(0, K, body, jnp.zeros((RB, K), jnp.int32))


def _dist_topk(x, xt, sq, sqt, batch_col, batch_row):
    return pl.pallas_call(
        _dist_body,
        grid=(NRB,),
        in_specs=[
            pl.BlockSpec((RB, ROI), lambda i: (i, 0)),
            pl.BlockSpec((ROI, N), lambda i: (0, 0)),
            pl.BlockSpec((RB, 1), lambda i: (i, 0)),
            pl.BlockSpec((1, N), lambda i: (0, 0)),
            pl.BlockSpec((RB, 1), lambda i: (i, 0)),
            pl.BlockSpec((1, N), lambda i: (0, 0)),
        ],
        out_specs=pl.BlockSpec((RB, K), lambda i: (i, 0)),
        out_shape=jax.ShapeDtypeStruct((N, K), jnp.int32),
        scratch_shapes=[pltpu.VMEM((RB, N), jnp.float32)],
    )(x, xt, sq, sqt, batch_col, batch_row)


def _scatter_body(dst_sm, xd_ref, xs_ref, w1_ref, b1_ref, w2_ref, b2_ref,
                  w3_ref, b3_ref, out_ref, s_ref, *accs):
    i = pl.program_id(0)

    @pl.when(i == 0)
    def _():
        for a in accs:
            a[...] = jnp.full((N, 48), -jnp.inf, jnp.float32)

    xi = xd_ref[:, :ROI]
    h = jnp.concatenate([xi, xs_ref[:, :ROI] - xi], axis=1)
    t1 = _lrelu(jnp.dot(h, w1_ref[...], preferred_element_type=jnp.float32)
                + b1_ref[...])
    m1 = jnp.dot(t1, w2_ref[...], preferred_element_type=jnp.float32) + b2_ref[...]
    m3 = jnp.dot(h, w3_ref[...], preferred_element_type=jnp.float32) + b3_ref[...]
    s_ref[:, :32] = m1
    s_ref[:, 32:] = jnp.broadcast_to(m3, (EB, 16))
    base = i * EB

    def body(r, carry):
        r8 = r * UNROLL
        for u in range(UNROLL):
            d = dst_sm[base + r8 + u]
            row = s_ref[pl.ds(r8 + u, 1), :]
            accs[u][pl.ds(d, 1), :] = jnp.maximum(accs[u][pl.ds(d, 1), :], row)
        return carry

    lax.fori_loop(0, EB // UNROLL, body, 0)

    @pl.when(i == NEB - 1)
    def _():
        m = accs[0][...]
        for a in accs[1:]:
            m = jnp.maximum(m, a[...])
        out_ref[...] = jnp.where(jnp.isfinite(m), m, 0.0)


def _edge_scatter(dst, xd, xs, w1, b1, w2, b2, w3, b3):
    grid_spec = pltpu.PrefetchScalarGridSpec(
        num_scalar_prefetch=1,
        grid=(NEB,),
        in_specs=[
            pl.BlockSpec((EB, 384), lambda i, *_: (i, 0)),
            pl.BlockSpec((EB, 384), lambda i, *_: (i, 0)),
            pl.BlockSpec((2 * ROI, 32), lambda i, *_: (0, 0)),
            pl.BlockSpec((1, 32), lambda i, *_: (0, 0)),
            pl.BlockSpec((32, 32), lambda i, *_: (0, 0)),
            pl.BlockSpec((1, 32), lambda i, *_: (0, 0)),
            pl.BlockSpec((2 * ROI, 1), lambda i, *_: (0, 0)),
            pl.BlockSpec((1, 1), lambda i, *_: (0, 0)),
        ],
        out_specs=pl.BlockSpec((N, 48), lambda i, *_: (0, 0)),
        scratch_shapes=[pltpu.VMEM((EB, 48), jnp.float32)]
        + [pltpu.VMEM((N, 48), jnp.float32) for _ in range(UNROLL)],
    )
    return pl.pallas_call(
        _scatter_body,
        grid_spec=grid_spec,
        out_shape=jax.ShapeDtypeStruct((N, 48), jnp.float32),
        compiler_params=pltpu.CompilerParams(
            dimension_semantics=("arbitrary",)),
    )(dst, xd, xs, w1, b1, w2, b2, w3, b3)


def _knn_body(b2g_ref, a2_ref, w_ref, b_ref, out_ref):
    b2 = b2g_ref[:, 96:128]
    a2 = a2_ref[...]
    a2r = jnp.broadcast_to(a2[:, None, :], (KNN_NODES, K, C)).reshape(KNN_NODES * K, C)
    m = jnp.dot(_lrelu(b2 + a2r), w_ref[...],
                preferred_element_type=jnp.float32) + b_ref[...]
    out_ref[...] = jnp.max(m.reshape(KNN_NODES, K, C), axis=1)


def _knn_max(b2g, a2, w, b):
    return pl.pallas_call(
        _knn_body,
        grid=(KNB,),
        in_specs=[
            pl.BlockSpec((KNN_NODES * K, 128), lambda i: (i, 0)),
            pl.BlockSpec((KNN_NODES, C), lambda i: (i, 0)),
            pl.BlockSpec((C, C), lambda i: (0, 0)),
            pl.BlockSpec((1, C), lambda i: (0, 0)),
        ],
        out_specs=pl.BlockSpec((KNN_NODES, C), lambda i: (i, 0)),
        out_shape=jax.ShapeDtypeStruct((N, C), jnp.float32),
    )(b2g, a2, w, b)


def _final1_body(acc_ref, x2_ref, bc_ref,
                 g1_ref, b1_ref, g2_ref, b2_ref, g3_ref, b3_ref,
                 p1_ref, p2_ref, x3_ref):
    accm = acc_ref[...]
    x1 = _lrelu(_bn(accm[:, :32], g1_ref[...], b1_ref[...]))
    x2 = _lrelu(_bn(x2_ref[...], g2_ref[...], b2_ref[...]))
    x3 = _lrelu(_bn(accm[:, 32:33], g3_ref[...], b3_ref[...]))
    oh = (lax.broadcasted_iota(jnp.int32, (B, N), 0) == bc_ref[...]).astype(jnp.float32)
    counts = jnp.maximum(jnp.sum(oh, axis=1, keepdims=True), 1.0)
    p1_ref[...] = jnp.dot(oh, x1, preferred_element_type=jnp.float32) / counts
    p2_ref[...] = jnp.dot(oh, x2, preferred_element_type=jnp.float32) / counts
    x3_ref[...] = x3


def _final1(acc, x2raw, batch_row, g1, b1, g2, b2, g3, b3):
    return pl.pallas_call(
        _final1_body,
        out_shape=(jax.ShapeDtypeStruct((B, C), jnp.float32),
                   jax.ShapeDtypeStruct((B, C), jnp.float32),
                   jax.ShapeDtypeStruct((N, 1), jnp.float32)),
    )(acc, x2raw, batch_row, g1, b1, g2, b2, g3, b3)


def _final2_body(h_ref, w1_ref, b1_ref, g1_ref, bb1_ref,
                 w2_ref, b2_ref, g2_ref, bb2_ref, w3_ref, b3_ref, out_ref):
    h = h_ref[...]
    h = _lrelu(_bn(jnp.dot(h, w1_ref[...], preferred_element_type=jnp.float32)
                   + b1_ref[...], g1_ref[...], bb1_ref[...]))
    h = _lrelu(_bn(jnp.dot(h, w2_ref[...], preferred_element_type=jnp.float32)
                   + b2_ref[...], g2_ref[...], bb2_ref[...]))
    out_ref[...] = jnp.dot(h, w3_ref[...], preferred_element_type=jnp.float32) + b3_ref[...]


def _final2(h, w1, b1, g1, bb1, w2, b2, g2, bb2, w3, b3):
    return pl.pallas_call(
        _final2_body,
        out_shape=jax.ShapeDtypeStruct((B, 1), jnp.float32),
    )(h, w1, b1, g1, bb1, w2, b2, g2, bb2, w3, b3)


NW = 32
EPW = E // NW
ECH = EPW // 4
KPW = (N * K) // NW
KCH = KPW // 8


FB = 128
NFB = E // FB


def _sc_gather_feats(xp, dst, src):
    mesh = plsc.VectorSubcoreMesh(core_axis_name="c", subcore_axis_name="s")

    @functools.partial(
        pl.kernel, mesh=mesh,
        out_type=[jax.ShapeDtypeStruct((E, 384), jnp.float32),
                  jax.ShapeDtypeStruct((E, 384), jnp.float32)],
        scratch_types=[
            pltpu.VMEM((FB,), jnp.int32),
            pltpu.VMEM((FB, 384), jnp.float32),
            pltpu.VMEM((FB,), jnp.int32),
            pltpu.VMEM((FB, 384), jnp.float32),
            pltpu.SemaphoreType.DMA,
            pltpu.SemaphoreType.DMA,
        ],
    )
    def gather_k(xp_h, dst_h, src_h, xd_h, xs_h,
                 idx_v, rows_v, idx2_v, rows2_v, sem, sem2):
        wid = lax.axis_index("s") * 2 + lax.axis_index("c")
        nblk = (NFB + NW - 1) // NW

        def step(k, carry):
            blk = wid + k * NW

            @pl.when(blk < NFB)
            def _():
                base = blk * FB
                pltpu.sync_copy(dst_h.at[pl.ds(base, FB)], idx_v)
                pltpu.sync_copy(src_h.at[pl.ds(base, FB)], idx2_v)
                cp1 = pltpu.async_copy(xp_h.at[idx_v], rows_v, sem)
                cp2 = pltpu.async_copy(xp_h.at[idx2_v], rows2_v, sem2)
                cp1.wait()
                pltpu.sync_copy(rows_v, xd_h.at[pl.ds(base, FB)])
                cp2.wait()
                pltpu.sync_copy(rows2_v, xs_h.at[pl.ds(base, FB)])
            return carry

        lax.fori_loop(0, nblk, step, 0)

    return gather_k(xp, dst, src)


def _sc_gather_knn(t128, idxk):
    mesh = plsc.VectorSubcoreMesh(core_axis_name="c", subcore_axis_name="s")

    @functools.partial(
        pl.kernel, mesh=mesh,
        out_type=jax.ShapeDtypeStruct((N * K, 128), jnp.float32),
        scratch_types=[
            pltpu.VMEM((KCH,), jnp.int32),
            pltpu.VMEM((KCH, 128), jnp.float32),
            pltpu.SemaphoreType.DMA,
        ],
    )
    def gather_k(t_h, idxk_h, b2g_h, idx_v, rows_v, sem):
        wid = lax.axis_index("s") * 2 + lax.axis_index("c")
        for ch in range(8):
            base = wid * KPW + ch * KCH
            pltpu.sync_copy(idxk_h.at[pl.ds(base, KCH)], idx_v)
            pltpu.async_copy(t_h.at[idx_v], rows_v, sem).wait()
            pltpu.sync_copy(rows_v, b2g_h.at[pl.ds(base, KCH)])

    return gather_k(t128, idxk)


def kernel(x, edge_index, batch, mlp1_w1, mlp1_b1, mlp1_w2, mlp1_b2, gbn1_g, gbn1_b,
           mlp2_w1, mlp2_b1, mlp2_w2, mlp2_b2, gbn2_g, gbn2_b,
           mlp3_w, mlp3_b, gbn3_g, gbn3_b,
           lin1_w, lin1_b, fbn1_g, fbn1_b, lin2_w, lin2_b, fbn2_g, fbn2_b,
           lin3_w, lin3_b):
    f32 = jnp.float32
    src = edge_index[0]
    dst = edge_index[1]

    z15 = jnp.zeros((ROI, 15), f32)
    w128 = jnp.concatenate([
        mlp1_w1[:ROI] - mlp1_w1[ROI:],
        mlp3_w[:ROI] - mlp3_w[ROI:],
        z15,
        mlp1_w1[ROI:],
        mlp3_w[ROI:],
        z15,
        mlp2_w1[ROI:],
    ], axis=1)
    b128 = jnp.concatenate([
        mlp1_b1, mlp3_b, jnp.zeros((15,), f32),
        jnp.zeros((48,), f32),
        jnp.zeros((32,), f32),
    ])[None, :]
    wa2 = mlp2_w1[:ROI] - mlp2_w1[ROI:]
    ba2 = mlp2_b1[None, :]
    xp = jnp.pad(x, ((0, 0), (0, 384 - ROI)))

    t128, a2 = _proj(x, w128, b128, wa2, ba2)
    sq = jnp.sum(x * x, axis=1)[:, None]

    batch_row = batch[None, :].astype(jnp.int32)
    idx = _dist_topk(x, x.T, sq, sq.T, batch[:, None].astype(jnp.int32), batch_row)

    xd, xs = _sc_gather_feats(xp, dst, src)
    b2g = _sc_gather_knn(t128, idx.reshape(-1))

    acc = _edge_scatter(dst, xd, xs, mlp1_w1, mlp1_b1[None, :], mlp1_w2,
                        mlp1_b2[None, :], mlp3_w, mlp3_b[None, :])

    x2raw = _knn_max(b2g, a2, mlp2_w2, mlp2_b2[None, :])

    p1, p2, x3 = _final1(acc, x2raw, batch_row,
                         gbn1_g[None, :], gbn1_b[None, :],
                         gbn2_g[None, :], gbn2_b[None, :],
                         gbn3_g[None, :], gbn3_b[None, :])

    h0 = jnp.concatenate([p1, p2, x3.reshape(B, ROI)], axis=1)

    return _final2(h0, lin1_w, lin1_b[None, :], fbn1_g[None, :], fbn1_b[None, :],
                   lin2_w, lin2_b[None, :], fbn2_g[None, :], fbn2_b[None, :],
                   lin3_w, lin3_b[None, :])


# speedup vs baseline: 8.3747x; 1.0754x over previous
import functools

import jax
import jax.numpy as jnp
from jax import lax
from jax.experimental import pallas as pl
from jax.experimental.pallas import tpu as pltpu
from jax.experimental.pallas import tpu_sc as plsc

N = 4288
B = 16
ROI = 268
C = 32
K = 32
E = 68608
SLOPE = 0.33
EPS = 1e-5

RB = 536
NRB = N // RB
EB = 2144
NEB = E // EB
UNROLL = 16
KNN_NODES = 64
KNB = N // KNN_NODES

MASK_INF = 1e30
PICK_INF = 2e30


def _lrelu(x):
    return jnp.where(x >= 0, x, SLOPE * x)


def _bn(x, g, b):
    mu = jnp.mean(x, axis=0, keepdims=True)
    var = jnp.mean((x - mu) ** 2, axis=0, keepdims=True)
    return (x - mu) * jax.lax.rsqrt(var + EPS) * g + b


def _proj_body(x_ref, w_ref, b_ref, w2_ref, b2_ref, p_ref, a2_ref):
    x = x_ref[...]
    p_ref[...] = jnp.dot(x, w_ref[...], precision=lax.Precision.HIGHEST,
                         preferred_element_type=jnp.float32) + b_ref[...]
    a2_ref[...] = jnp.dot(x, w2_ref[...], precision=lax.Precision.HIGHEST,
                          preferred_element_type=jnp.float32) + b2_ref[...]


def _proj(x, w128, b128, wa2, ba2):
    return pl.pallas_call(
        _proj_body,
        out_shape=(jax.ShapeDtypeStruct((N, 128), jnp.float32),
                   jax.ShapeDtypeStruct((N, C), jnp.float32)),
    )(x, w128, b128, wa2, ba2)


CW = 256
NCW = (N + CW - 1) // CW
NPAD = NCW * CW


def _dist_body(x_ref, xt_ref, sqr_ref, sqc_ref, br_ref, bc_ref, idx_ref, d_ref):
    xb = x_ref[...]
    d = sqr_ref[...] + sqc_ref[...] - 2.0 * jnp.dot(
        xb, xt_ref[...], preferred_element_type=jnp.float32)
    d = jnp.where(br_ref[...] == bc_ref[...], d, MASK_INF)
    d_ref[:, :N] = d
    d_ref[:, N:] = jnp.full((RB, NPAD - N), PICK_INF, jnp.float32)
    col = lax.broadcasted_iota(jnp.int32, (RB, N), 1)
    colc = lax.broadcasted_iota(jnp.int32, (RB, CW), 1)
    bmin = br_ref[0, 0]
    bmax = br_ref[RB - 1, 0]
    bc = bc_ref[...]
    lo = jnp.sum(jnp.where(bc < bmin, 1, 0))
    hi = jnp.sum(jnp.where(bc <= bmax, 1, 0))
    clo = lo // CW
    chi = (hi + CW - 1) // CW

    def body(k, acc):
        def p1(c, m):
            dch = d_ref[:, pl.ds(c * CW, CW)]
            return jnp.minimum(m, jnp.min(dch, axis=1, keepdims=True))

        m = lax.fori_loop(clo, chi, p1, jnp.full((RB, 1), jnp.inf, jnp.float32))
        exhausted = jnp.max(jnp.where(m >= MASK_INF, 1, 0)) > 0

        def win_branch():
            def p2(c, cand):
                dch = d_ref[:, pl.ds(c * CW, CW)]
                cc = jnp.where(dch == m, colc + c * CW, N)
                return jnp.minimum(cand, jnp.min(cc, axis=1, keepdims=True))

            sel = lax.fori_loop(clo, chi, p2, jnp.full((RB, 1), N, jnp.int32))

            def p3(c, carry):
                dch = d_ref[:, pl.ds(c * CW, CW)]
                d_ref[:, pl.ds(c * CW, CW)] = jnp.where(
                    colc + c * CW == sel, PICK_INF, dch)
                return carry

            lax.fori_loop(clo, chi, p3, 0)
            return sel

        def full_branch():
            dcur = d_ref[:, :N]
            mf = jnp.min(dcur, axis=1, keepdims=True)
            self_sel = jnp.min(jnp.where(dcur == mf, col, N), axis=1,
                               keepdims=True)
            d_ref[:, :N] = jnp.where(col == self_sel, PICK_INF, dcur)
            return self_sel

        sel = lax.cond(exhausted, full_branch, win_branch)
        acc = jnp.where(lax.broadcasted_iota(jnp.int32, (RB, K), 1) == k, sel, acc)
        return acc

    idx_ref2 = lax.fori_loop(0, K, body, jnp.zeros((RB, K), jnp.int32))
    idx_ref[...] = idx_ref2


def _dist_topk(x, xt, sq, sqt, batch_col, batch_row):
    return pl.pallas_call(
        _dist_body,
        grid=(NRB,),
        in_specs=[
            pl.BlockSpec((RB, ROI), lambda i: (i, 0)),
            pl.BlockSpec((ROI, N), lambda i: (0, 0)),
            pl.BlockSpec((RB, 1), lambda i: (i, 0)),
            pl.BlockSpec((1, N), lambda i: (0, 0)),
            pl.BlockSpec((RB, 1), lambda i: (i, 0)),
            pl.BlockSpec((1, N), lambda i: (0, 0)),
        ],
        out_specs=pl.BlockSpec((RB, K), lambda i: (i, 0)),
        out_shape=jax.ShapeDtypeStruct((N, K), jnp.int32),
        scratch_shapes=[pltpu.VMEM((RB, NPAD), jnp.float32)],
    )(x, xt, sq, sqt, batch_col, batch_row)


def _scatter_body(dst_sm, xd_ref, xs_ref, w1_ref, b1_ref, w2_ref, b2_ref,
                  w3_ref, b3_ref, out_ref, s_ref, *accs):
    i = pl.program_id(0)

    @pl.when(i == 0)
    def _():
        for a in accs:
            a[...] = jnp.full((N, 48), -jnp.inf, jnp.float32)

    xi = xd_ref[:, :ROI]
    h = jnp.concatenate([xi, xs_ref[:, :ROI] - xi], axis=1)
    t1 = _lrelu(jnp.dot(h, w1_ref[...], preferred_element_type=jnp.float32)
                + b1_ref[...])
    m1 = jnp.dot(t1, w2_ref[...], preferred_element_type=jnp.float32) + b2_ref[...]
    m3 = jnp.dot(h, w3_ref[...], preferred_element_type=jnp.float32) + b3_ref[...]
    s_ref[:, :32] = m1
    s_ref[:, 32:] = jnp.broadcast_to(m3, (EB, 16))
    base = i * EB

    def body(r, carry):
        r8 = r * UNROLL
        for u in range(UNROLL):
            d = dst_sm[base + r8 + u]
            row = s_ref[pl.ds(r8 + u, 1), :]
            accs[u][pl.ds(d, 1), :] = jnp.maximum(accs[u][pl.ds(d, 1), :], row)
        return carry

    lax.fori_loop(0, EB // UNROLL, body, 0)

    @pl.when(i == NEB - 1)
    def _():
        m = accs[0][...]
        for a in accs[1:]:
            m = jnp.maximum(m, a[...])
        out_ref[...] = jnp.where(jnp.isfinite(m), m, 0.0)


def _edge_scatter(dst, xd, xs, w1, b1, w2, b2, w3, b3):
    grid_spec = pltpu.PrefetchScalarGridSpec(
        num_scalar_prefetch=1,
        grid=(NEB,),
        in_specs=[
            pl.BlockSpec((EB, 384), lambda i, *_: (i, 0)),
            pl.BlockSpec((EB, 384), lambda i, *_: (i, 0)),
            pl.BlockSpec((2 * ROI, 32), lambda i, *_: (0, 0)),
            pl.BlockSpec((1, 32), lambda i, *_: (0, 0)),
            pl.BlockSpec((32, 32), lambda i, *_: (0, 0)),
            pl.BlockSpec((1, 32), lambda i, *_: (0, 0)),
            pl.BlockSpec((2 * ROI, 1), lambda i, *_: (0, 0)),
            pl.BlockSpec((1, 1), lambda i, *_: (0, 0)),
        ],
        out_specs=pl.BlockSpec((N, 48), lambda i, *_: (0, 0)),
        scratch_shapes=[pltpu.VMEM((EB, 48), jnp.float32)]
        + [pltpu.VMEM((N, 48), jnp.float32) for _ in range(UNROLL)],
    )
    return pl.pallas_call(
        _scatter_body,
        grid_spec=grid_spec,
        out_shape=jax.ShapeDtypeStruct((N, 48), jnp.float32),
        compiler_params=pltpu.CompilerParams(
            dimension_semantics=("arbitrary",)),
    )(dst, xd, xs, w1, b1, w2, b2, w3, b3)


def _knn_body(b2g_ref, a2_ref, w_ref, b_ref, out_ref):
    b2 = b2g_ref[:, 96:128]
    a2 = a2_ref[...]
    a2r = jnp.broadcast_to(a2[:, None, :], (KNN_NODES, K, C)).reshape(KNN_NODES * K, C)
    m = jnp.dot(_lrelu(b2 + a2r), w_ref[...],
                preferred_element_type=jnp.float32) + b_ref[...]
    out_ref[...] = jnp.max(m.reshape(KNN_NODES, K, C), axis=1)


def _knn_max(b2g, a2, w, b):
    return pl.pallas_call(
        _knn_body,
        grid=(KNB,),
        in_specs=[
            pl.BlockSpec((KNN_NODES * K, 128), lambda i: (i, 0)),
            pl.BlockSpec((KNN_NODES, C), lambda i: (i, 0)),
            pl.BlockSpec((C, C), lambda i: (0, 0)),
            pl.BlockSpec((1, C), lambda i: (0, 0)),
        ],
        out_specs=pl.BlockSpec((KNN_NODES, C), lambda i: (i, 0)),
        out_shape=jax.ShapeDtypeStruct((N, C), jnp.float32),
    )(b2g, a2, w, b)


def _final1_body(acc_ref, x2_ref, bc_ref,
                 g1_ref, b1_ref, g2_ref, b2_ref, g3_ref, b3_ref,
                 p1_ref, p2_ref, x3_ref):
    accm = acc_ref[...]
    x1 = _lrelu(_bn(accm[:, :32], g1_ref[...], b1_ref[...]))
    x2 = _lrelu(_bn(x2_ref[...], g2_ref[...], b2_ref[...]))
    x3 = _lrelu(_bn(accm[:, 32:33], g3_ref[...], b3_ref[...]))
    oh = (lax.broadcasted_iota(jnp.int32, (B, N), 0) == bc_ref[...]).astype(jnp.float32)
    counts = jnp.maximum(jnp.sum(oh, axis=1, keepdims=True), 1.0)
    p1_ref[...] = jnp.dot(oh, x1, preferred_element_type=jnp.float32) / counts
    p2_ref[...] = jnp.dot(oh, x2, preferred_element_type=jnp.float32) / counts
    x3_ref[...] = x3


def _final1(acc, x2raw, batch_row, g1, b1, g2, b2, g3, b3):
    return pl.pallas_call(
        _final1_body,
        out_shape=(jax.ShapeDtypeStruct((B, C), jnp.float32),
                   jax.ShapeDtypeStruct((B, C), jnp.float32),
                   jax.ShapeDtypeStruct((N, 1), jnp.float32)),
    )(acc, x2raw, batch_row, g1, b1, g2, b2, g3, b3)


def _final2_body(h_ref, w1_ref, b1_ref, g1_ref, bb1_ref,
                 w2_ref, b2_ref, g2_ref, bb2_ref, w3_ref, b3_ref, out_ref):
    h = h_ref[...]
    h = _lrelu(_bn(jnp.dot(h, w1_ref[...], preferred_element_type=jnp.float32)
                   + b1_ref[...], g1_ref[...], bb1_ref[...]))
    h = _lrelu(_bn(jnp.dot(h, w2_ref[...], preferred_element_type=jnp.float32)
                   + b2_ref[...], g2_ref[...], bb2_ref[...]))
    out_ref[...] = jnp.dot(h, w3_ref[...], preferred_element_type=jnp.float32) + b3_ref[...]


def _final2(h, w1, b1, g1, bb1, w2, b2, g2, bb2, w3, b3):
    return pl.pallas_call(
        _final2_body,
        out_shape=jax.ShapeDtypeStruct((B, 1), jnp.float32),
    )(h, w1, b1, g1, bb1, w2, b2, g2, bb2, w3, b3)


NW = 32
EPW = E // NW
ECH = EPW // 4
KPW = (N * K) // NW
KCH = KPW // 8


FB = 128
NFB = E // FB


def _sc_gather_feats(xp, dst, src):
    mesh = plsc.VectorSubcoreMesh(core_axis_name="c", subcore_axis_name="s")

    @functools.partial(
        pl.kernel, mesh=mesh,
        out_type=[jax.ShapeDtypeStruct((E, 384), jnp.float32),
                  jax.ShapeDtypeStruct((E, 384), jnp.float32)],
        scratch_types=[
            pltpu.VMEM((FB,), jnp.int32),
            pltpu.VMEM((FB, 384), jnp.float32),
            pltpu.VMEM((FB,), jnp.int32),
            pltpu.VMEM((FB, 384), jnp.float32),
            pltpu.SemaphoreType.DMA,
            pltpu.SemaphoreType.DMA,
        ],
    )
    def gather_k(xp_h, dst_h, src_h, xd_h, xs_h,
                 idx_v, rows_v, idx2_v, rows2_v, sem, sem2):
        wid = lax.axis_index("s") * 2 + lax.axis_index("c")
        nblk = (NFB + NW - 1) // NW

        def step(k, carry):
            blk = wid + k * NW

            @pl.when(blk < NFB)
            def _():
                base = blk * FB
                pltpu.sync_copy(dst_h.at[pl.ds(base, FB)], idx_v)
                pltpu.sync_copy(src_h.at[pl.ds(base, FB)], idx2_v)
                cp1 = pltpu.async_copy(xp_h.at[idx_v], rows_v, sem)
                cp2 = pltpu.async_copy(xp_h.at[idx2_v], rows2_v, sem2)
                cp1.wait()
                pltpu.sync_copy(rows_v, xd_h.at[pl.ds(base, FB)])
                cp2.wait()
                pltpu.sync_copy(rows2_v, xs_h.at[pl.ds(base, FB)])
            return carry

        lax.fori_loop(0, nblk, step, 0)

    return gather_k(xp, dst, src)


def _sc_gather_knn(t128, idxk):
    mesh = plsc.VectorSubcoreMesh(core_axis_name="c", subcore_axis_name="s")

    @functools.partial(
        pl.kernel, mesh=mesh,
        out_type=jax.ShapeDtypeStruct((N * K, 128), jnp.float32),
        scratch_types=[
            pltpu.VMEM((KCH,), jnp.int32),
            pltpu.VMEM((KCH, 128), jnp.float32),
            pltpu.SemaphoreType.DMA,
        ],
    )
    def gather_k(t_h, idxk_h, b2g_h, idx_v, rows_v, sem):
        wid = lax.axis_index("s") * 2 + lax.axis_index("c")
        for ch in range(8):
            base = wid * KPW + ch * KCH
            pltpu.sync_copy(idxk_h.at[pl.ds(base, KCH)], idx_v)
            pltpu.async_copy(t_h.at[idx_v], rows_v, sem).wait()
            pltpu.sync_copy(rows_v, b2g_h.at[pl.ds(base, KCH)])

    return gather_k(t128, idxk)


def kernel(x, edge_index, batch, mlp1_w1, mlp1_b1, mlp1_w2, mlp1_b2, gbn1_g, gbn1_b,
           mlp2_w1, mlp2_b1, mlp2_w2, mlp2_b2, gbn2_g, gbn2_b,
           mlp3_w, mlp3_b, gbn3_g, gbn3_b,
           lin1_w, lin1_b, fbn1_g, fbn1_b, lin2_w, lin2_b, fbn2_g, fbn2_b,
           lin3_w, lin3_b):
    f32 = jnp.float32
    src = edge_index[0]
    dst = edge_index[1]

    z15 = jnp.zeros((ROI, 15), f32)
    w128 = jnp.concatenate([
        mlp1_w1[:ROI] - mlp1_w1[ROI:],
        mlp3_w[:ROI] - mlp3_w[ROI:],
        z15,
        mlp1_w1[ROI:],
        mlp3_w[ROI:],
        z15,
        mlp2_w1[ROI:],
    ], axis=1)
    b128 = jnp.concatenate([
        mlp1_b1, mlp3_b, jnp.zeros((15,), f32),
        jnp.zeros((48,), f32),
        jnp.zeros((32,), f32),
    ])[None, :]
    wa2 = mlp2_w1[:ROI] - mlp2_w1[ROI:]
    ba2 = mlp2_b1[None, :]
    xp = jnp.pad(x, ((0, 0), (0, 384 - ROI)))

    t128, a2 = _proj(x, w128, b128, wa2, ba2)
    sq = jnp.sum(x * x, axis=1)[:, None]

    batch_row = batch[None, :].astype(jnp.int32)
    idx = _dist_topk(x, x.T, sq, sq.T, batch[:, None].astype(jnp.int32), batch_row)

    xd, xs = _sc_gather_feats(xp, dst, src)
    b2g = _sc_gather_knn(t128, idx.reshape(-1))

    acc = _edge_scatter(dst, xd, xs, mlp1_w1, mlp1_b1[None, :], mlp1_w2,
                        mlp1_b2[None, :], mlp3_w, mlp3_b[None, :])

    x2raw = _knn_max(b2g, a2, mlp2_w2, mlp2_b2[None, :])

    p1, p2, x3 = _final1(acc, x2raw, batch_row,
                         gbn1_g[None, :], gbn1_b[None, :],
                         gbn2_g[None, :], gbn2_b[None, :],
                         gbn3_g[None, :], gbn3_b[None, :])

    h0 = jnp.concatenate([p1, p2, x3.reshape(B, ROI)], axis=1)

    return _final2(h0, lin1_w, lin1_b[None, :], fbn1_g[None, :], fbn1_b[None, :],
                   lin2_w, lin2_b[None, :], fbn2_g[None, :], fbn2_b[None, :],
                   lin3_w, lin3_b[None, :])
